# Initial kernel scaffold; baseline (speedup 1.0000x reference)
import jax
import jax.numpy as jnp
from jax.experimental import pallas as pl


def kernel(x, edge_index, edge_attr, batch, W0, b0, We1, be1, We2, be2, Wn1, bn1, Wn2, bn2, conv_b, Wg_ih, Wg_hh, bg_ih, bg_hh, Ws_ih, Ws_hh, bs_ih, bs_hh, W1, b1, W2, b2):
    raise NotImplementedError("write your pallas kernel here")


# speedup vs baseline: 3.7379x; 3.7379x over previous
import functools

import jax
import jax.numpy as jnp
from jax import lax
from jax.experimental import pallas as pl
from jax.experimental.pallas import tpu as pltpu
from jax.experimental.pallas import tpu_sc as plsc

N = 10000
E = 160000
DN = 128
DE = 16
H = 16
EH = 64
BG = 64
HH = H * H

NC = 2
NS = 16
NW = NC * NS
EPW = E // NW
CH = 125
CPW = EPW // CH
RPT = N // NS


def _gather_body(table, idx, out, idx_v, rows_v):
  cid = lax.axis_index("c")
  sid = lax.axis_index("s")
  wid = cid * NS + sid
  pltpu.sync_copy(idx.at[pl.ds(wid * CPW, CPW)], idx_v)

  def body(j, carry):
    pltpu.sync_copy(table.at[idx_v.at[j]], rows_v.at[pl.ds(j * CH, CH)])
    return carry

  lax.fori_loop(0, CPW, body, 0)
  pltpu.sync_copy(rows_v, out.at[pl.ds(wid * EPW, EPW)])


@functools.cache
def _sc_mesh():
  return plsc.VectorSubcoreMesh(
      core_axis_name="c", subcore_axis_name="s",
      num_cores=NC, num_subcores=NS)


_SC_PARAMS = pltpu.CompilerParams(use_tc_tiling_on_sc=False)


@functools.cache
def _sc_gather_kernel():
  return pl.kernel(
      _gather_body,
      out_type=jax.ShapeDtypeStruct((E, H), jnp.float32),
      mesh=_sc_mesh(),
      scratch_types=[
          pltpu.VMEM((CPW, CH), jnp.int32),
          pltpu.VMEM((EPW, H), jnp.float32),
      ],
      compiler_params=_SC_PARAMS,
  )


def _sc_gather(table, idx):
  return _sc_gather_kernel()(table, idx)


def _scatter_body(msg, idx, zeros, s_out, idx_v, rows_v, acc):
  cid = lax.axis_index("c")
  sid = lax.axis_index("s")
  wid = cid * NS + sid
  pltpu.sync_copy(idx.at[pl.ds(wid * CPW, CPW)], idx_v)
  pltpu.sync_copy(msg.at[pl.ds(wid * EPW, EPW)], rows_v)

  @pl.when(sid == 0)
  def _():
    pltpu.sync_copy(zeros, acc)

  plsc.subcore_barrier()

  def body(j, carry):
    pltpu.sync_copy(rows_v.at[pl.ds(j * CH, CH)], acc.at[idx_v.at[j]],
                    add=True)
    return carry

  lax.fori_loop(0, CPW, body, 0)
  plsc.subcore_barrier()
  base = sid * RPT
  pltpu.sync_copy(acc.at[pl.ds(base, RPT)],
                  s_out.at[cid].at[pl.ds(base, RPT)])


@functools.cache
def _sc_scatter_kernel():
  return pl.kernel(
      _scatter_body,
      out_type=jax.ShapeDtypeStruct((NC, N, H), jnp.float32),
      mesh=_sc_mesh(),
      scratch_types=[
          pltpu.VMEM((CPW, CH), jnp.int32),
          pltpu.VMEM((EPW, H), jnp.float32),
          pltpu.VMEM_SHARED((N, H), jnp.float32),
      ],
      compiler_params=_SC_PARAMS,
  )


def _sc_scatter(msg, idx, zeros):
  return _sc_scatter_kernel()(msg, idx, zeros)


def _scatter_cnt_body(msg, idx, zeros, ones, s_out, c_out,
                      idx_v, rows_v, ones_v, acc, cacc):
  cid = lax.axis_index("c")
  sid = lax.axis_index("s")
  wid = cid * NS + sid
  pltpu.sync_copy(idx.at[pl.ds(wid * CPW, CPW)], idx_v)
  pltpu.sync_copy(msg.at[pl.ds(wid * EPW, EPW)], rows_v)
  pltpu.sync_copy(ones, ones_v)

  @pl.when(sid == 0)
  def _():
    pltpu.sync_copy(zeros, acc)
    pltpu.sync_copy(zeros, cacc)

  plsc.subcore_barrier()

  def body(j, carry):
    idxrow = idx_v.at[j]
    pltpu.sync_copy(rows_v.at[pl.ds(j * CH, CH)], acc.at[idxrow], add=True)
    pltpu.sync_copy(ones_v, cacc.at[idxrow], add=True)
    return carry

  lax.fori_loop(0, CPW, body, 0)
  plsc.subcore_barrier()
  base = sid * RPT
  pltpu.sync_copy(acc.at[pl.ds(base, RPT)],
                  s_out.at[cid].at[pl.ds(base, RPT)])
  pltpu.sync_copy(cacc.at[pl.ds(base, RPT)],
                  c_out.at[cid].at[pl.ds(base, RPT)])


@functools.cache
def _sc_scatter_cnt_kernel():
  return pl.kernel(
      _scatter_cnt_body,
      out_type=(
          jax.ShapeDtypeStruct((NC, N, H), jnp.float32),
          jax.ShapeDtypeStruct((NC, N, H), jnp.float32),
      ),
      mesh=_sc_mesh(),
      scratch_types=[
          pltpu.VMEM((CPW, CH), jnp.int32),
          pltpu.VMEM((EPW, H), jnp.float32),
          pltpu.VMEM((CH, H), jnp.float32),
          pltpu.VMEM_SHARED((N, H), jnp.float32),
          pltpu.VMEM_SHARED((N, H), jnp.float32),
      ],
      compiler_params=_SC_PARAMS,
  )


def _sc_scatter_cnt(msg, idx, zeros, ones):
  return _sc_scatter_cnt_kernel()(msg, idx, zeros, ones)


EB = 2000
EG = E // EB


def _edge_block_math(ea, xj, we1, be1, we2, be2, emat, cred):
  hid = jnp.maximum(
      jnp.dot(ea, we1, preferred_element_type=jnp.float32) + be1, 0.0)
  ew = jnp.dot(hid, we2, preferred_element_type=jnp.float32) + be2
  xe = jnp.dot(xj, emat, preferred_element_type=jnp.float32)
  msg = jnp.dot(ew * xe, cred, preferred_element_type=jnp.float32)
  return ew, msg


def _edge_body(ea_ref, xj_ref, we1, be1, we2, be2, emat, cred, msg_ref):
  _, msg = _edge_block_math(ea_ref[...], xj_ref[...], we1[...], be1[...],
                            we2[...], be2[...], emat[...], cred[...])
  msg_ref[...] = msg


def _edge_fg_body(ea_ref, xj_ref, we1, be1, we2, be2, emat, cred,
                  wn1, bn1, wn2, bn2, msg_ref, fg_ref):
  ea = ea_ref[...]
  ew, msg = _edge_block_math(ea, xj_ref[...], we1[...], be1[...],
                             we2[...], be2[...], emat[...], cred[...])
  msg_ref[...] = msg
  rec = jnp.maximum(
      jnp.dot(ew, wn1[...], preferred_element_type=jnp.float32) + bn1[...],
      0.0)
  rec = jnp.dot(rec, wn2[...], preferred_element_type=jnp.float32) + bn2[...]
  d = rec - ea

  @pl.when(pl.program_id(0) == 0)
  def _():
    fg_ref[...] = jnp.zeros_like(fg_ref)

  fg_ref[...] += jnp.sum(d * d)[None, None]


def _full(shape):
  return pl.BlockSpec(shape, lambda i: tuple(0 for _ in shape))


def _edge_specs():
  return [
      pl.BlockSpec((EB, DE), lambda i: (i, 0)),
      pl.BlockSpec((EB, H), lambda i: (i, 0)),
      _full((DE, EH)), _full((1, EH)),
      _full((EH, HH)), _full((1, HH)),
      _full((H, HH)),
      _full((HH, H)),
  ]


_tc_edge = pl.pallas_call(
    _edge_body,
    grid=(EG,),
    in_specs=_edge_specs(),
    out_specs=pl.BlockSpec((EB, H), lambda i: (i, 0)),
    out_shape=jax.ShapeDtypeStruct((E, H), jnp.float32),
)

_tc_edge_fg = pl.pallas_call(
    _edge_fg_body,
    grid=(EG,),
    in_specs=_edge_specs() + [
        _full((HH, EH)), _full((1, EH)),
        _full((EH, DE)), _full((1, DE)),
    ],
    out_specs=(
        pl.BlockSpec((EB, H), lambda i: (i, 0)),
        pl.BlockSpec((1, 1), lambda i: (0, 0)),
    ),
    out_shape=(
        jax.ShapeDtypeStruct((E, H), jnp.float32),
        jax.ShapeDtypeStruct((1, 1), jnp.float32),
    ),
)


def _init_body(x_ref, w0, b0, out_ref):
  out_ref[...] = jnp.maximum(
      jnp.dot(x_ref[...], w0[...], preferred_element_type=jnp.float32)
      + b0[...], 0.0)


_tc_init = pl.pallas_call(
    _init_body,
    grid=(1,),
    in_specs=[_full((N, DN)), _full((DN, H)), _full((1, H))],
    out_specs=pl.BlockSpec((N, H), lambda i: (0, 0)),
    out_shape=jax.ShapeDtypeStruct((N, H), jnp.float32),
)


def _node_body(sp_ref, cp_ref, h_ref, convb, wgit, bgi, wght, bgh, h_out):
  s = sp_ref[0] + sp_ref[1]
  cnt = cp_ref[0, :, 0:1] + cp_ref[1, :, 0:1]
  m = jnp.maximum(s / jnp.maximum(cnt, 1.0) + convb[...], 0.0)
  gi = jnp.dot(m, wgit[...], preferred_element_type=jnp.float32) + bgi[...]
  h = h_ref[...]
  gh = jnp.dot(h, wght[...], preferred_element_type=jnp.float32) + bgh[...]
  r = jax.nn.sigmoid(gi[:, :H] + gh[:, :H])
  z = jax.nn.sigmoid(gi[:, H:2 * H] + gh[:, H:2 * H])
  n = jnp.tanh(gi[:, 2 * H:] + r * gh[:, 2 * H:])
  h_out[...] = (1.0 - z) * n + z * h


_tc_node = pl.pallas_call(
    _node_body,
    grid=(1,),
    in_specs=[
        _full((NC, N, H)), _full((NC, N, H)), _full((N, H)),
        _full((1, H)), _full((H, 3 * H)), _full((1, 3 * H)),
        _full((H, 3 * H)), _full((1, 3 * H)),
    ],
    out_specs=pl.BlockSpec((N, H), lambda i: (0, 0)),
    out_shape=jax.ShapeDtypeStruct((N, H), jnp.float32),
)


def _s2s_body(out_ref, batch_ref, batch_t_ref, wsit, bsi, wsht, bsh,
              w1, b1, w2, b2, og_ref):
  out = out_ref[...]
  bcol = batch_ref[...]
  brow = batch_t_ref[...]
  iota_col = lax.broadcasted_iota(jnp.int32, (N, BG), 1)
  iota_row = lax.broadcasted_iota(jnp.int32, (BG, N), 0)
  onehot = (bcol == iota_col).astype(jnp.float32)
  onehot_t = (brow == iota_row).astype(jnp.float32)

  hs = jnp.zeros((BG, H), jnp.float32)
  cs = jnp.zeros((BG, H), jnp.float32)
  q_star = jnp.zeros((BG, 2 * H), jnp.float32)
  neg_inf = jnp.float32(-jnp.inf)

  for _ in range(3):
    gates = (jnp.dot(q_star, wsit[...], preferred_element_type=jnp.float32)
             + bsi[...]
             + jnp.dot(hs, wsht[...], preferred_element_type=jnp.float32)
             + bsh[...])
    i_g = jax.nn.sigmoid(gates[:, :H])
    f_g = jax.nn.sigmoid(gates[:, H:2 * H])
    g_g = jnp.tanh(gates[:, 2 * H:3 * H])
    o_g = jax.nn.sigmoid(gates[:, 3 * H:])
    cs = f_g * cs + i_g * g_g
    hs = o_g * jnp.tanh(cs)
    q = hs
    qb = jnp.dot(onehot, q, preferred_element_type=jnp.float32)
    e = jnp.sum(out * qb, axis=1, keepdims=True)
    emat = jnp.where(onehot_t > 0.0, e.reshape(1, N), neg_inf)
    emax = jnp.max(emat, axis=1, keepdims=True)
    emax = jnp.where(emax > neg_inf, emax, 0.0)
    a = jnp.exp(e - jnp.dot(onehot, emax,
                            preferred_element_type=jnp.float32))
    asum = jnp.dot(onehot_t, a, preferred_element_type=jnp.float32)
    a = a / jnp.maximum(jnp.dot(onehot, asum,
                                preferred_element_type=jnp.float32), 1e-16)
    rvec = jnp.dot(onehot_t, a * out, preferred_element_type=jnp.float32)
    q_star = jnp.concatenate([q, rvec], axis=-1)

  og = jnp.maximum(
      jnp.dot(q_star, w1[...], preferred_element_type=jnp.float32) + b1[...],
      0.0)
  og_ref[...] = jnp.dot(og, w2[...],
                        preferred_element_type=jnp.float32) + b2[...]


_tc_s2s = pl.pallas_call(
    _s2s_body,
    grid=(1,),
    in_specs=[
        _full((N, H)), _full((N, 1)), _full((1, N)),
        _full((2 * H, 4 * H)), _full((1, 4 * H)),
        _full((H, 4 * H)), _full((1, 4 * H)),
        _full((2 * H, H)), _full((1, H)),
        _full((H, 12)), _full((1, 12)),
    ],
    out_specs=pl.BlockSpec((BG, 12), lambda i: (0, 0)),
    out_shape=jax.ShapeDtypeStruct((BG, 12), jnp.float32),
)


@jax.jit
def kernel(x, edge_index, edge_attr, batch, W0, b0, We1, be1, We2, be2,
           Wn1, bn1, Wn2, bn2, conv_b, Wg_ih, Wg_hh, bg_ih, bg_hh,
           Ws_ih, Ws_hh, bs_ih, bs_hh, W1, b1, W2, b2):
  f32 = jnp.float32
  src = edge_index[0].reshape(E // CH, CH)
  dst = edge_index[1].reshape(E // CH, CH)

  jidx = jnp.arange(HH, dtype=jnp.int32)
  emat = (jidx[None, :] // H == jnp.arange(H, dtype=jnp.int32)[:, None])
  emat = emat.astype(f32)
  cred = (jidx[:, None] % H == jnp.arange(H, dtype=jnp.int32)[None, :])
  cred = cred.astype(f32)

  zeros_nh = jnp.zeros((N, H), f32)
  ones_ch = jnp.ones((CH, H), f32)

  out = _tc_init(x, W0, b0.reshape(1, H))
  h = out
  fg = None
  cnt_p = None
  for step in range(3):
    xj = _sc_gather(out, src)
    if step == 0:
      msg, fg_acc = _tc_edge_fg(
          edge_attr, xj, We1, be1.reshape(1, EH), We2, be2.reshape(1, HH),
          emat, cred, Wn1, bn1.reshape(1, EH), Wn2, bn2.reshape(1, DE))
      fg = fg_acc[0, 0] / f32(E * DE)
      s_p, cnt_p = _sc_scatter_cnt(msg, dst, zeros_nh, ones_ch)
    else:
      msg = _tc_edge(edge_attr, xj, We1, be1.reshape(1, EH), We2,
                     be2.reshape(1, HH), emat, cred)
      s_p = _sc_scatter(msg, dst, zeros_nh)
    h = _tc_node(s_p, cnt_p, h, conv_b.reshape(1, H),
                 Wg_ih.T, bg_ih.reshape(1, 3 * H),
                 Wg_hh.T, bg_hh.reshape(1, 3 * H))
    out = h

  og = _tc_s2s(out, batch.reshape(N, 1), batch.reshape(1, N),
               Ws_ih.T, bs_ih.reshape(1, 4 * H),
               Ws_hh.T, bs_hh.reshape(1, 4 * H),
               W1, b1.reshape(1, H), W2, b2.reshape(1, 12))
  return og, fg


# speedup vs baseline: 4.0143x; 1.0740x over previous
import functools

import jax
import jax.numpy as jnp
from jax import lax
from jax.experimental import pallas as pl
from jax.experimental.pallas import tpu as pltpu
from jax.experimental.pallas import tpu_sc as plsc

N = 10000
E = 160000
DN = 128
DE = 16
H = 16
EH = 64
BG = 64
HH = H * H

NC = 2
NS = 16
NW = NC * NS
EPW = E // NW
CH = 125
CPW = EPW // CH
RPT = N // NS


def _gather_body(table, idx, out, idx_v, rows_v):
  cid = lax.axis_index("c")
  sid = lax.axis_index("s")
  wid = cid * NS + sid
  pltpu.sync_copy(idx.at[pl.ds(wid * CPW, CPW)], idx_v)

  def body(j, carry):
    pltpu.sync_copy(table.at[idx_v.at[j]], rows_v.at[pl.ds(j * CH, CH)])
    return carry

  lax.fori_loop(0, CPW, body, 0)
  pltpu.sync_copy(rows_v, out.at[pl.ds(wid * EPW, EPW)])


@functools.cache
def _sc_mesh():
  return plsc.VectorSubcoreMesh(
      core_axis_name="c", subcore_axis_name="s",
      num_cores=NC, num_subcores=NS)


_SC_PARAMS = pltpu.CompilerParams(use_tc_tiling_on_sc=False)


@functools.cache
def _sc_gather_kernel():
  return pl.kernel(
      _gather_body,
      out_type=jax.ShapeDtypeStruct((E, H), jnp.float32),
      mesh=_sc_mesh(),
      scratch_types=[
          pltpu.VMEM((CPW, CH), jnp.int32),
          pltpu.VMEM((EPW, H), jnp.float32),
      ],
      compiler_params=_SC_PARAMS,
  )


def _sc_gather(table, idx):
  return _sc_gather_kernel()(table, idx)


def _scatter_body(msg, idx, zeros, s_out, idx_v, rows_v, acc):
  cid = lax.axis_index("c")
  sid = lax.axis_index("s")
  wid = cid * NS + sid
  pltpu.sync_copy(idx.at[pl.ds(wid * CPW, CPW)], idx_v)
  pltpu.sync_copy(msg.at[pl.ds(wid * EPW, EPW)], rows_v)

  @pl.when(sid == 0)
  def _():
    pltpu.sync_copy(zeros, acc)

  plsc.subcore_barrier()

  def body(j, carry):
    pltpu.sync_copy(rows_v.at[pl.ds(j * CH, CH)], acc.at[idx_v.at[j]],
                    add=True)
    return carry

  lax.fori_loop(0, CPW, body, 0)
  plsc.subcore_barrier()
  base = sid * RPT
  pltpu.sync_copy(acc.at[pl.ds(base, RPT)],
                  s_out.at[cid].at[pl.ds(base, RPT)])


@functools.cache
def _sc_scatter_kernel():
  return pl.kernel(
      _scatter_body,
      out_type=jax.ShapeDtypeStruct((NC, N, H), jnp.float32),
      mesh=_sc_mesh(),
      scratch_types=[
          pltpu.VMEM((CPW, CH), jnp.int32),
          pltpu.VMEM((EPW, H), jnp.float32),
          pltpu.VMEM_SHARED((N, H), jnp.float32),
      ],
      compiler_params=_SC_PARAMS,
  )


def _sc_scatter(msg, idx, zeros):
  return _sc_scatter_kernel()(msg, idx, zeros)


def _scatter_cnt_body(msg, idx, zeros, ones, s_out, c_out,
                      idx_v, rows_v, ones_v, acc, cacc):
  cid = lax.axis_index("c")
  sid = lax.axis_index("s")
  wid = cid * NS + sid
  pltpu.sync_copy(idx.at[pl.ds(wid * CPW, CPW)], idx_v)
  pltpu.sync_copy(msg.at[pl.ds(wid * EPW, EPW)], rows_v)
  pltpu.sync_copy(ones, ones_v)

  @pl.when(sid == 0)
  def _():
    pltpu.sync_copy(zeros, acc)
    pltpu.sync_copy(zeros, cacc)

  plsc.subcore_barrier()

  def body(j, carry):
    idxrow = idx_v.at[j]
    pltpu.sync_copy(rows_v.at[pl.ds(j * CH, CH)], acc.at[idxrow], add=True)
    pltpu.sync_copy(ones_v, cacc.at[idxrow], add=True)
    return carry

  lax.fori_loop(0, CPW, body, 0)
  plsc.subcore_barrier()
  base = sid * RPT
  pltpu.sync_copy(acc.at[pl.ds(base, RPT)],
                  s_out.at[cid].at[pl.ds(base, RPT)])
  pltpu.sync_copy(cacc.at[pl.ds(base, RPT)],
                  c_out.at[cid].at[pl.ds(base, RPT)])


@functools.cache
def _sc_scatter_cnt_kernel():
  return pl.kernel(
      _scatter_cnt_body,
      out_type=(
          jax.ShapeDtypeStruct((NC, N, H), jnp.float32),
          jax.ShapeDtypeStruct((NC, N, H), jnp.float32),
      ),
      mesh=_sc_mesh(),
      scratch_types=[
          pltpu.VMEM((CPW, CH), jnp.int32),
          pltpu.VMEM((EPW, H), jnp.float32),
          pltpu.VMEM((CH, H), jnp.float32),
          pltpu.VMEM_SHARED((N, H), jnp.float32),
          pltpu.VMEM_SHARED((N, H), jnp.float32),
      ],
      compiler_params=_SC_PARAMS,
  )


def _sc_scatter_cnt(msg, idx, zeros, ones):
  return _sc_scatter_cnt_kernel()(msg, idx, zeros, ones)


EB = 4000
EG = E // EB

_BF = jnp.bfloat16


def _edge_block_math(ea, xj, we1, be1, we2, be2, emat, cred):
  hid = jnp.maximum(
      jnp.dot(ea.astype(_BF), we1.astype(_BF),
              preferred_element_type=jnp.float32) + be1, 0.0)
  ew = jnp.dot(hid.astype(_BF), we2.astype(_BF),
               preferred_element_type=jnp.float32) + be2
  xe = jnp.dot(xj.astype(_BF), emat.astype(_BF),
               preferred_element_type=jnp.float32)
  msg = jnp.dot((ew * xe).astype(_BF), cred.astype(_BF),
                preferred_element_type=jnp.float32)
  return ew, msg


def _edge_body(ea_ref, xj_ref, we1, be1, we2, be2, emat, cred, msg_ref):
  _, msg = _edge_block_math(ea_ref[...], xj_ref[...], we1[...], be1[...],
                            we2[...], be2[...], emat[...], cred[...])
  msg_ref[...] = msg


def _edge_fg_body(ea_ref, xj_ref, we1, be1, we2, be2, emat, cred,
                  wn1, bn1, wn2, bn2, msg_ref, fg_ref):
  ea = ea_ref[...]
  ew, msg = _edge_block_math(ea, xj_ref[...], we1[...], be1[...],
                             we2[...], be2[...], emat[...], cred[...])
  msg_ref[...] = msg
  rec = jnp.maximum(
      jnp.dot(ew, wn1[...], preferred_element_type=jnp.float32) + bn1[...],
      0.0)
  rec = jnp.dot(rec, wn2[...], preferred_element_type=jnp.float32) + bn2[...]
  d = rec - ea

  @pl.when(pl.program_id(0) == 0)
  def _():
    fg_ref[...] = jnp.zeros_like(fg_ref)

  fg_ref[...] += jnp.sum(d * d)[None, None]


def _full(shape):
  return pl.BlockSpec(shape, lambda i: tuple(0 for _ in shape))


def _edge_specs():
  return [
      pl.BlockSpec((EB, DE), lambda i: (i, 0)),
      pl.BlockSpec((EB, H), lambda i: (i, 0)),
      _full((DE, EH)), _full((1, EH)),
      _full((EH, HH)), _full((1, HH)),
      _full((H, HH)),
      _full((HH, H)),
  ]


_tc_edge = pl.pallas_call(
    _edge_body,
    grid=(EG,),
    in_specs=_edge_specs(),
    out_specs=pl.BlockSpec((EB, H), lambda i: (i, 0)),
    out_shape=jax.ShapeDtypeStruct((E, H), jnp.float32),
)

_tc_edge_fg = pl.pallas_call(
    _edge_fg_body,
    grid=(EG,),
    in_specs=_edge_specs() + [
        _full((HH, EH)), _full((1, EH)),
        _full((EH, DE)), _full((1, DE)),
    ],
    out_specs=(
        pl.BlockSpec((EB, H), lambda i: (i, 0)),
        pl.BlockSpec((1, 1), lambda i: (0, 0)),
    ),
    out_shape=(
        jax.ShapeDtypeStruct((E, H), jnp.float32),
        jax.ShapeDtypeStruct((1, 1), jnp.float32),
    ),
)


def _init_body(x_ref, w0, b0, out_ref):
  out_ref[...] = jnp.maximum(
      jnp.dot(x_ref[...], w0[...], preferred_element_type=jnp.float32)
      + b0[...], 0.0)


_tc_init = pl.pallas_call(
    _init_body,
    grid=(1,),
    in_specs=[_full((N, DN)), _full((DN, H)), _full((1, H))],
    out_specs=pl.BlockSpec((N, H), lambda i: (0, 0)),
    out_shape=jax.ShapeDtypeStruct((N, H), jnp.float32),
)


def _node_body(sp_ref, cp_ref, h_ref, convb, wgit, bgi, wght, bgh, h_out):
  h_out[...] = _gru_math(sp_ref, cp_ref, h_ref[...], convb[...], wgit[...],
                         bgi[...], wght[...], bgh[...])


_tc_node = pl.pallas_call(
    _node_body,
    grid=(1,),
    in_specs=[
        _full((NC, N, H)), _full((NC, N, H)), _full((N, H)),
        _full((1, H)), _full((H, 3 * H)), _full((1, 3 * H)),
        _full((H, 3 * H)), _full((1, 3 * H)),
    ],
    out_specs=pl.BlockSpec((N, H), lambda i: (0, 0)),
    out_shape=jax.ShapeDtypeStruct((N, H), jnp.float32),
)


def _gru_math(sp_ref, cp_ref, h, convb, wgit, bgi, wght, bgh):
  s = sp_ref[0] + sp_ref[1]
  cnt = cp_ref[0, :, 0:1] + cp_ref[1, :, 0:1]
  m = jnp.maximum(s / jnp.maximum(cnt, 1.0) + convb, 0.0)
  gi = jnp.dot(m, wgit, preferred_element_type=jnp.float32) + bgi
  gh = jnp.dot(h, wght, preferred_element_type=jnp.float32) + bgh
  r = jax.nn.sigmoid(gi[:, :H] + gh[:, :H])
  z = jax.nn.sigmoid(gi[:, H:2 * H] + gh[:, H:2 * H])
  n = jnp.tanh(gi[:, 2 * H:] + r * gh[:, 2 * H:])
  return (1.0 - z) * n + z * h


def _s2s_body(sp_ref, cp_ref, h_ref, convb, wgit, bgi, wght, bgh,
              batch_ref, batch_t_ref, wsit, bsi, wsht, bsh,
              w1, b1, w2, b2, og_ref):
  out = _gru_math(sp_ref, cp_ref, h_ref[...], convb[...], wgit[...],
                  bgi[...], wght[...], bgh[...])
  bcol = batch_ref[...]
  brow = batch_t_ref[...]
  iota_col = lax.broadcasted_iota(jnp.int32, (N, BG), 1)
  iota_row = lax.broadcasted_iota(jnp.int32, (BG, N), 0)
  onehot = (bcol == iota_col).astype(jnp.float32)
  onehot_t = (brow == iota_row).astype(jnp.float32)

  hs = jnp.zeros((BG, H), jnp.float32)
  cs = jnp.zeros((BG, H), jnp.float32)
  q_star = jnp.zeros((BG, 2 * H), jnp.float32)
  neg_inf = jnp.float32(-jnp.inf)

  for _ in range(3):
    gates = (jnp.dot(q_star, wsit[...], preferred_element_type=jnp.float32)
             + bsi[...]
             + jnp.dot(hs, wsht[...], preferred_element_type=jnp.float32)
             + bsh[...])
    i_g = jax.nn.sigmoid(gates[:, :H])
    f_g = jax.nn.sigmoid(gates[:, H:2 * H])
    g_g = jnp.tanh(gates[:, 2 * H:3 * H])
    o_g = jax.nn.sigmoid(gates[:, 3 * H:])
    cs = f_g * cs + i_g * g_g
    hs = o_g * jnp.tanh(cs)
    q = hs
    qb = jnp.dot(onehot, q, preferred_element_type=jnp.float32)
    e = jnp.sum(out * qb, axis=1, keepdims=True)
    emat = jnp.where(onehot_t > 0.0, e.reshape(1, N), neg_inf)
    emax = jnp.max(emat, axis=1, keepdims=True)
    emax = jnp.where(emax > neg_inf, emax, 0.0)
    a = jnp.exp(e - jnp.dot(onehot, emax,
                            preferred_element_type=jnp.float32))
    asum = jnp.dot(onehot_t, a, preferred_element_type=jnp.float32)
    a = a / jnp.maximum(jnp.dot(onehot, asum,
                                preferred_element_type=jnp.float32), 1e-16)
    rvec = jnp.dot(onehot_t, a * out, preferred_element_type=jnp.float32)
    q_star = jnp.concatenate([q, rvec], axis=-1)

  og = jnp.maximum(
      jnp.dot(q_star, w1[...], preferred_element_type=jnp.float32) + b1[...],
      0.0)
  og_ref[...] = jnp.dot(og, w2[...],
                        preferred_element_type=jnp.float32) + b2[...]


_tc_s2s = pl.pallas_call(
    _s2s_body,
    grid=(1,),
    in_specs=[
        _full((NC, N, H)), _full((NC, N, H)), _full((N, H)),
        _full((1, H)), _full((H, 3 * H)), _full((1, 3 * H)),
        _full((H, 3 * H)), _full((1, 3 * H)),
        _full((N, 1)), _full((1, N)),
        _full((2 * H, 4 * H)), _full((1, 4 * H)),
        _full((H, 4 * H)), _full((1, 4 * H)),
        _full((2 * H, H)), _full((1, H)),
        _full((H, 12)), _full((1, 12)),
    ],
    out_specs=pl.BlockSpec((BG, 12), lambda i: (0, 0)),
    out_shape=jax.ShapeDtypeStruct((BG, 12), jnp.float32),
)


@jax.jit
def kernel(x, edge_index, edge_attr, batch, W0, b0, We1, be1, We2, be2,
           Wn1, bn1, Wn2, bn2, conv_b, Wg_ih, Wg_hh, bg_ih, bg_hh,
           Ws_ih, Ws_hh, bs_ih, bs_hh, W1, b1, W2, b2):
  f32 = jnp.float32
  src = edge_index[0].reshape(E // CH, CH)
  dst = edge_index[1].reshape(E // CH, CH)

  jidx = jnp.arange(HH, dtype=jnp.int32)
  emat = (jidx[None, :] // H == jnp.arange(H, dtype=jnp.int32)[:, None])
  emat = emat.astype(f32)
  cred = (jidx[:, None] % H == jnp.arange(H, dtype=jnp.int32)[None, :])
  cred = cred.astype(f32)

  zeros_nh = jnp.zeros((N, H), f32)
  ones_ch = jnp.ones((CH, H), f32)

  gru_args = (conv_b.reshape(1, H), Wg_ih.T, bg_ih.reshape(1, 3 * H),
              Wg_hh.T, bg_hh.reshape(1, 3 * H))

  out = _tc_init(x, W0, b0.reshape(1, H))
  h = out
  fg = None
  cnt_p = None
  for step in range(3):
    xj = _sc_gather(out, src)
    if step == 0:
      msg, fg_acc = _tc_edge_fg(
          edge_attr, xj, We1, be1.reshape(1, EH), We2, be2.reshape(1, HH),
          emat, cred, Wn1, bn1.reshape(1, EH), Wn2, bn2.reshape(1, DE))
      fg = fg_acc[0, 0] / f32(E * DE)
      s_p, cnt_p = _sc_scatter_cnt(msg, dst, zeros_nh, ones_ch)
    else:
      msg = _tc_edge(edge_attr, xj, We1, be1.reshape(1, EH), We2,
                     be2.reshape(1, HH), emat, cred)
      s_p = _sc_scatter(msg, dst, zeros_nh)
    if step < 2:
      h = _tc_node(s_p, cnt_p, h, *gru_args)
      out = h

  og = _tc_s2s(s_p, cnt_p, h, *gru_args,
               batch.reshape(N, 1), batch.reshape(1, N),
               Ws_ih.T, bs_ih.reshape(1, 4 * H),
               Ws_hh.T, bs_hh.reshape(1, 4 * H),
               W1, b1.reshape(1, H), W2, b2.reshape(1, 12))
  return og, fg


# speedup vs baseline: 4.2180x; 1.0507x over previous
import functools

import jax
import jax.numpy as jnp
from jax import lax
from jax.experimental import pallas as pl
from jax.experimental.pallas import tpu as pltpu
from jax.experimental.pallas import tpu_sc as plsc

N = 10000
E = 160000
DN = 128
DE = 16
H = 16
EH = 64
BG = 64
HH = H * H

NC = 2
NS = 16
NW = NC * NS
EPW = E // NW
CH = 125
CPW = EPW // CH
RPT = N // NS


def _gather_body(table, idx, out, idx_v, rows_v, tab_s):
  cid = lax.axis_index("c")
  sid = lax.axis_index("s")
  wid = cid * NS + sid

  @pl.when(sid == 0)
  def _():
    pltpu.sync_copy(table, tab_s)

  pltpu.sync_copy(idx.at[pl.ds(wid * CPW, CPW)], idx_v)
  plsc.subcore_barrier()

  def body(j, carry):
    pltpu.sync_copy(tab_s.at[idx_v.at[j]], rows_v.at[pl.ds(j * CH, CH)])
    return carry

  lax.fori_loop(0, CPW, body, 0)

  pltpu.sync_copy(rows_v, out.at[pl.ds(wid * EPW, EPW)])


@functools.cache
def _sc_mesh():
  return plsc.VectorSubcoreMesh(
      core_axis_name="c", subcore_axis_name="s",
      num_cores=NC, num_subcores=NS)


_SC_PARAMS = pltpu.CompilerParams(use_tc_tiling_on_sc=False)


@functools.cache
def _sc_gather_kernel():
  return pl.kernel(
      _gather_body,
      out_type=jax.ShapeDtypeStruct((E, H), jnp.float32),
      mesh=_sc_mesh(),
      scratch_types=[
          pltpu.VMEM((CPW, CH), jnp.int32),
          pltpu.VMEM((EPW, H), jnp.float32),
          pltpu.VMEM_SHARED((N, H), jnp.float32),
      ],
      compiler_params=_SC_PARAMS,
  )


def _sc_gather(table, idx):
  return _sc_gather_kernel()(table, idx)


def _scatter_body(msg, idx, zeros, s_out, idx_v, rows_v, acc):
  cid = lax.axis_index("c")
  sid = lax.axis_index("s")
  wid = cid * NS + sid
  pltpu.sync_copy(idx.at[pl.ds(wid * CPW, CPW)], idx_v)
  pltpu.sync_copy(msg.at[pl.ds(wid * EPW, EPW)], rows_v)

  @pl.when(sid == 0)
  def _():
    pltpu.sync_copy(zeros, acc)

  plsc.subcore_barrier()

  def body(j, carry):
    pltpu.sync_copy(rows_v.at[pl.ds(j * CH, CH)], acc.at[idx_v.at[j]],
                    add=True)
    return carry

  lax.fori_loop(0, CPW, body, 0)

  plsc.subcore_barrier()
  base = sid * RPT
  pltpu.sync_copy(acc.at[pl.ds(base, RPT)],
                  s_out.at[cid].at[pl.ds(base, RPT)])


@functools.cache
def _sc_scatter_kernel():
  return pl.kernel(
      _scatter_body,
      out_type=jax.ShapeDtypeStruct((NC, N, H), jnp.float32),
      mesh=_sc_mesh(),
      scratch_types=[
          pltpu.VMEM((CPW, CH), jnp.int32),
          pltpu.VMEM((EPW, H), jnp.float32),
          pltpu.VMEM_SHARED((N, H), jnp.float32),
      ],
      compiler_params=_SC_PARAMS,
  )


def _sc_scatter(msg, idx, zeros):
  return _sc_scatter_kernel()(msg, idx, zeros)


def _scatter_cnt_body(msg, idx, zeros, ones, s_out, c_out,
                      idx_v, rows_v, ones_v, acc, cacc):
  cid = lax.axis_index("c")
  sid = lax.axis_index("s")
  wid = cid * NS + sid
  pltpu.sync_copy(idx.at[pl.ds(wid * CPW, CPW)], idx_v)
  pltpu.sync_copy(msg.at[pl.ds(wid * EPW, EPW)], rows_v)
  pltpu.sync_copy(ones, ones_v)

  @pl.when(sid == 0)
  def _():
    pltpu.sync_copy(zeros, acc)
    pltpu.sync_copy(zeros, cacc)

  plsc.subcore_barrier()

  def body(j, carry):
    idxrow = idx_v.at[j]
    pltpu.sync_copy(rows_v.at[pl.ds(j * CH, CH)], acc.at[idxrow], add=True)
    pltpu.sync_copy(ones_v, cacc.at[idxrow], add=True)
    return carry

  lax.fori_loop(0, CPW, body, 0)

  plsc.subcore_barrier()
  base = sid * RPT
  pltpu.sync_copy(acc.at[pl.ds(base, RPT)],
                  s_out.at[cid].at[pl.ds(base, RPT)])
  pltpu.sync_copy(cacc.at[pl.ds(base, RPT)],
                  c_out.at[cid].at[pl.ds(base, RPT)])


@functools.cache
def _sc_scatter_cnt_kernel():
  return pl.kernel(
      _scatter_cnt_body,
      out_type=(
          jax.ShapeDtypeStruct((NC, N, H), jnp.float32),
          jax.ShapeDtypeStruct((NC, N, H), jnp.float32),
      ),
      mesh=_sc_mesh(),
      scratch_types=[
          pltpu.VMEM((CPW, CH), jnp.int32),
          pltpu.VMEM((EPW, H), jnp.float32),
          pltpu.VMEM((CH, H), jnp.float32),
          pltpu.VMEM_SHARED((N, H), jnp.float32),
          pltpu.VMEM_SHARED((N, H), jnp.float32),
      ],
      compiler_params=_SC_PARAMS,
  )


def _sc_scatter_cnt(msg, idx, zeros, ones):
  return _sc_scatter_cnt_kernel()(msg, idx, zeros, ones)


EB = 4000
EG = E // EB

_BF = jnp.bfloat16


def _edge_block_math(ea, xj, we1, be1, we2, be2, emat, cred):
  hid = jnp.maximum(
      jnp.dot(ea.astype(_BF), we1.astype(_BF),
              preferred_element_type=jnp.float32) + be1, 0.0)
  ew = jnp.dot(hid.astype(_BF), we2.astype(_BF),
               preferred_element_type=jnp.float32) + be2
  xe = jnp.dot(xj.astype(_BF), emat.astype(_BF),
               preferred_element_type=jnp.float32)
  msg = jnp.dot((ew * xe).astype(_BF), cred.astype(_BF),
                preferred_element_type=jnp.float32)
  return ew, msg


def _edge_body(ea_ref, xj_ref, we1, be1, we2, be2, emat, cred, msg_ref):
  _, msg = _edge_block_math(ea_ref[...], xj_ref[...], we1[...], be1[...],
                            we2[...], be2[...], emat[...], cred[...])
  msg_ref[...] = msg


def _edge_fg_body(ea_ref, xj_ref, we1, be1, we2, be2, emat, cred,
                  wn1, bn1, wn2, bn2, msg_ref, fg_ref):
  ea = ea_ref[...]
  ew, msg = _edge_block_math(ea, xj_ref[...], we1[...], be1[...],
                             we2[...], be2[...], emat[...], cred[...])
  msg_ref[...] = msg
  rec = jnp.maximum(
      jnp.dot(ew, wn1[...], preferred_element_type=jnp.float32) + bn1[...],
      0.0)
  rec = jnp.dot(rec, wn2[...], preferred_element_type=jnp.float32) + bn2[...]
  d = rec - ea

  @pl.when(pl.program_id(0) == 0)
  def _():
    fg_ref[...] = jnp.zeros_like(fg_ref)

  fg_ref[...] += jnp.sum(d * d)[None, None]


def _full(shape):
  return pl.BlockSpec(shape, lambda i: tuple(0 for _ in shape))


def _edge_specs():
  return [
      pl.BlockSpec((EB, DE), lambda i: (i, 0)),
      pl.BlockSpec((EB, H), lambda i: (i, 0)),
      _full((DE, EH)), _full((1, EH)),
      _full((EH, HH)), _full((1, HH)),
      _full((H, HH)),
      _full((HH, H)),
  ]


_tc_edge = pl.pallas_call(
    _edge_body,
    grid=(EG,),
    in_specs=_edge_specs(),
    out_specs=pl.BlockSpec((EB, H), lambda i: (i, 0)),
    out_shape=jax.ShapeDtypeStruct((E, H), jnp.float32),
)

_tc_edge_fg = pl.pallas_call(
    _edge_fg_body,
    grid=(EG,),
    in_specs=_edge_specs() + [
        _full((HH, EH)), _full((1, EH)),
        _full((EH, DE)), _full((1, DE)),
    ],
    out_specs=(
        pl.BlockSpec((EB, H), lambda i: (i, 0)),
        pl.BlockSpec((1, 1), lambda i: (0, 0)),
    ),
    out_shape=(
        jax.ShapeDtypeStruct((E, H), jnp.float32),
        jax.ShapeDtypeStruct((1, 1), jnp.float32),
    ),
)


def _init_body(x_ref, w0, b0, out_ref):
  out_ref[...] = jnp.maximum(
      jnp.dot(x_ref[...], w0[...], preferred_element_type=jnp.float32)
      + b0[...], 0.0)


_tc_init = pl.pallas_call(
    _init_body,
    grid=(1,),
    in_specs=[_full((N, DN)), _full((DN, H)), _full((1, H))],
    out_specs=pl.BlockSpec((N, H), lambda i: (0, 0)),
    out_shape=jax.ShapeDtypeStruct((N, H), jnp.float32),
)


def _node_body(sp_ref, cp_ref, h_ref, convb, wgit, bgi, wght, bgh, h_out):
  h_out[...] = _gru_math(sp_ref, cp_ref, h_ref[...], convb[...], wgit[...],
                         bgi[...], wght[...], bgh[...])


_tc_node = pl.pallas_call(
    _node_body,
    grid=(1,),
    in_specs=[
        _full((NC, N, H)), _full((NC, N, H)), _full((N, H)),
        _full((1, H)), _full((H, 3 * H)), _full((1, 3 * H)),
        _full((H, 3 * H)), _full((1, 3 * H)),
    ],
    out_specs=pl.BlockSpec((N, H), lambda i: (0, 0)),
    out_shape=jax.ShapeDtypeStruct((N, H), jnp.float32),
)


def _gru_math(sp_ref, cp_ref, h, convb, wgit, bgi, wght, bgh):
  s = sp_ref[0] + sp_ref[1]
  cnt = cp_ref[0, :, 0:1] + cp_ref[1, :, 0:1]
  m = jnp.maximum(s / jnp.maximum(cnt, 1.0) + convb, 0.0)
  gi = jnp.dot(m, wgit, preferred_element_type=jnp.float32) + bgi
  gh = jnp.dot(h, wght, preferred_element_type=jnp.float32) + bgh
  r = jax.nn.sigmoid(gi[:, :H] + gh[:, :H])
  z = jax.nn.sigmoid(gi[:, H:2 * H] + gh[:, H:2 * H])
  n = jnp.tanh(gi[:, 2 * H:] + r * gh[:, 2 * H:])
  return (1.0 - z) * n + z * h


def _s2s_body(sp_ref, cp_ref, h_ref, convb, wgit, bgi, wght, bgh,
              batch_ref, batch_t_ref, wsit, bsi, wsht, bsh,
              w1, b1, w2, b2, og_ref):
  out = _gru_math(sp_ref, cp_ref, h_ref[...], convb[...], wgit[...],
                  bgi[...], wght[...], bgh[...])
  bcol = batch_ref[...]
  brow = batch_t_ref[...]
  iota_col = lax.broadcasted_iota(jnp.int32, (N, BG), 1)
  iota_row = lax.broadcasted_iota(jnp.int32, (BG, N), 0)
  onehot = (bcol == iota_col).astype(jnp.float32)
  onehot_t = (brow == iota_row).astype(jnp.float32)

  hs = jnp.zeros((BG, H), jnp.float32)
  cs = jnp.zeros((BG, H), jnp.float32)
  q_star = jnp.zeros((BG, 2 * H), jnp.float32)
  neg_inf = jnp.float32(-jnp.inf)

  for _ in range(3):
    gates = (jnp.dot(q_star, wsit[...], preferred_element_type=jnp.float32)
             + bsi[...]
             + jnp.dot(hs, wsht[...], preferred_element_type=jnp.float32)
             + bsh[...])
    i_g = jax.nn.sigmoid(gates[:, :H])
    f_g = jax.nn.sigmoid(gates[:, H:2 * H])
    g_g = jnp.tanh(gates[:, 2 * H:3 * H])
    o_g = jax.nn.sigmoid(gates[:, 3 * H:])
    cs = f_g * cs + i_g * g_g
    hs = o_g * jnp.tanh(cs)
    q = hs
    qb = jnp.dot(onehot, q, preferred_element_type=jnp.float32)
    e = jnp.sum(out * qb, axis=1, keepdims=True)
    emat = jnp.where(onehot_t > 0.0, e.reshape(1, N), neg_inf)
    emax = jnp.max(emat, axis=1, keepdims=True)
    emax = jnp.where(emax > neg_inf, emax, 0.0)
    a = jnp.exp(e - jnp.dot(onehot, emax,
                            preferred_element_type=jnp.float32))
    asum = jnp.dot(onehot_t, a, preferred_element_type=jnp.float32)
    a = a / jnp.maximum(jnp.dot(onehot, asum,
                                preferred_element_type=jnp.float32), 1e-16)
    rvec = jnp.dot(onehot_t, a * out, preferred_element_type=jnp.float32)
    q_star = jnp.concatenate([q, rvec], axis=-1)

  og = jnp.maximum(
      jnp.dot(q_star, w1[...], preferred_element_type=jnp.float32) + b1[...],
      0.0)
  og_ref[...] = jnp.dot(og, w2[...],
                        preferred_element_type=jnp.float32) + b2[...]


_tc_s2s = pl.pallas_call(
    _s2s_body,
    grid=(1,),
    in_specs=[
        _full((NC, N, H)), _full((NC, N, H)), _full((N, H)),
        _full((1, H)), _full((H, 3 * H)), _full((1, 3 * H)),
        _full((H, 3 * H)), _full((1, 3 * H)),
        _full((N, 1)), _full((1, N)),
        _full((2 * H, 4 * H)), _full((1, 4 * H)),
        _full((H, 4 * H)), _full((1, 4 * H)),
        _full((2 * H, H)), _full((1, H)),
        _full((H, 12)), _full((1, 12)),
    ],
    out_specs=pl.BlockSpec((BG, 12), lambda i: (0, 0)),
    out_shape=jax.ShapeDtypeStruct((BG, 12), jnp.float32),
)


@jax.jit
def kernel(x, edge_index, edge_attr, batch, W0, b0, We1, be1, We2, be2,
           Wn1, bn1, Wn2, bn2, conv_b, Wg_ih, Wg_hh, bg_ih, bg_hh,
           Ws_ih, Ws_hh, bs_ih, bs_hh, W1, b1, W2, b2):
  f32 = jnp.float32
  src = edge_index[0].reshape(E // CH, CH)
  dst = edge_index[1].reshape(E // CH, CH)

  jidx = jnp.arange(HH, dtype=jnp.int32)
  emat = (jidx[None, :] // H == jnp.arange(H, dtype=jnp.int32)[:, None])
  emat = emat.astype(f32)
  cred = (jidx[:, None] % H == jnp.arange(H, dtype=jnp.int32)[None, :])
  cred = cred.astype(f32)

  zeros_nh = jnp.zeros((N, H), f32)
  ones_ch = jnp.ones((CH, H), f32)

  gru_args = (conv_b.reshape(1, H), Wg_ih.T, bg_ih.reshape(1, 3 * H),
              Wg_hh.T, bg_hh.reshape(1, 3 * H))

  out = _tc_init(x, W0, b0.reshape(1, H))
  h = out
  fg = None
  cnt_p = None
  for step in range(3):
    xj = _sc_gather(out, src)
    if step == 0:
      msg, fg_acc = _tc_edge_fg(
          edge_attr, xj, We1, be1.reshape(1, EH), We2, be2.reshape(1, HH),
          emat, cred, Wn1, bn1.reshape(1, EH), Wn2, bn2.reshape(1, DE))
      fg = fg_acc[0, 0] / f32(E * DE)
      s_p, cnt_p = _sc_scatter_cnt(msg, dst, zeros_nh, ones_ch)
    else:
      msg = _tc_edge(edge_attr, xj, We1, be1.reshape(1, EH), We2,
                     be2.reshape(1, HH), emat, cred)
      s_p = _sc_scatter(msg, dst, zeros_nh)
    if step < 2:
      h = _tc_node(s_p, cnt_p, h, *gru_args)
      out = h

  og = _tc_s2s(s_p, cnt_p, h, *gru_args,
               batch.reshape(N, 1), batch.reshape(1, N),
               Ws_ih.T, bs_ih.reshape(1, 4 * H),
               Ws_hh.T, bs_hh.reshape(1, 4 * H),
               W1, b1.reshape(1, H), W2, b2.reshape(1, 12))
  return og, fg
